# baseline (speedup 1.0000x reference)
import functools

import jax
import jax.numpy as jnp
from jax import lax
from jax.experimental import pallas as pl
from jax.experimental.pallas import tpu as pltpu
from jax.experimental.pallas import tpu_sc as plsc

_NC = 2
_NS = 16
_NW = _NC * _NS


def _make_gather(S, B, V, D):
    chunk = B // _NW
    NG = chunk // 16
    CB = chunk // 128
    R = D // 8
    assert chunk * _NW == B and NG * 16 == chunk and CB * 128 == chunk
    mesh = plsc.VectorSubcoreMesh(core_axis_name="c", subcore_axis_name="s")

    @functools.partial(
        pl.kernel,
        mesh=mesh,
        out_type=jax.ShapeDtypeStruct((S, R, B // 128, 8, 128), jnp.float32),
        scratch_types=[
            pltpu.VMEM((S, chunk), jnp.int32),
            pltpu.VMEM((chunk, D), jnp.float32),
            pltpu.VMEM((chunk, D), jnp.float32),
            pltpu.VMEM((CB, D, 128), jnp.float32),
            pltpu.VMEM((CB, D, 128), jnp.float32),
            pltpu.SemaphoreType.DMA((2,)),
            pltpu.SemaphoreType.DMA((2,)),
        ],
        compiler_params=pltpu.CompilerParams(
            use_tc_tiling_on_sc=False, needs_layout_passes=False
        ),
    )
    def gather_kernel(table_hbm, idx_hbm, out_hbm, idx_v, r0, r1, t0, t1,
                      gsem, ssem):
        rows = [r0, r1]
        rowsT = [t0, t1]
        wid = lax.axis_index("s") * _NC + lax.axis_index("c")
        col0 = wid * chunk

        pltpu.sync_copy(idx_hbm.at[:, pl.ds(col0, chunk)], idx_v)

        lane16 = lax.iota(jnp.int32, 16)

        def gather_copy(s, b):
            return pltpu.make_async_copy(
                table_hbm.at[idx_v.at[s]], rows[b], gsem.at[b]
            )

        def store_copies(s, b):
            return [
                pltpu.make_async_copy(
                    rowsT[b].at[:, pl.ds(r * 8, 8), :],
                    out_hbm.at[s, r, pl.ds(wid * CB, CB)],
                    ssem.at[b],
                )
                for r in range(R)
            ]

        def transpose_chunk(b):
            src, dst = rows[b], rowsT[b]

            def body(g, carry):
                rowvec = g * 16 + lane16
                cb = g // 8
                l0 = (g % 8) * 16
                for d in range(D):
                    vals = plsc.load_gather(
                        src, [rowvec, jnp.full((16,), d, jnp.int32)]
                    )
                    dst[cb, d, pl.ds(l0, 16)] = vals
                return carry

            lax.fori_loop(0, NG, body, 0)

        gather_copy(0, 0).start()

        def outer(k, carry):
            for sub in range(2):
                s = 2 * k + sub
                b = sub
                gather_copy(s, b).wait()

                @pl.when(s + 1 < S)
                def _():
                    gather_copy(s + 1, 1 - b).start()

                @pl.when(k >= 1)
                def _():
                    for cp in store_copies(s - 2, b):
                        cp.wait()

                transpose_chunk(b)
                for cp in store_copies(s, b):
                    cp.start()
            return carry

        lax.fori_loop(0, S // 2, outer, 0)
        for sub in range(2):
            for cp in store_copies(S - 2 + sub, sub):
                cp.wait()

    return gather_kernel


def kernel(x, table):
    Bm, S = x.shape
    V, D = table.shape
    xt = jnp.swapaxes(x, 0, 1).astype(jnp.int32)
    y = _make_gather(S, Bm, V, D)(table, xt)
    return jnp.transpose(y, (2, 4, 0, 1, 3)).reshape(Bm, S, D)


# speedup vs baseline: 1.4450x; 1.4450x over previous
import functools

import jax
import jax.numpy as jnp
from jax import lax
from jax.experimental import pallas as pl
from jax.experimental.pallas import tpu as pltpu
from jax.experimental.pallas import tpu_sc as plsc

_NC = 2
_NS = 16
_NW = _NC * _NS


def _make_gather(S, B, V, D):
    chunk = B // _NW
    NG = chunk // 16
    CB = chunk // 128
    R = D // 8
    assert chunk * _NW == B and NG * 16 == chunk and CB * 128 == chunk
    mesh = plsc.VectorSubcoreMesh(core_axis_name="c", subcore_axis_name="s")

    @functools.partial(
        pl.kernel,
        mesh=mesh,
        out_type=jax.ShapeDtypeStruct((S, R, B // 128, 8, 128), jnp.float32),
        scratch_types=[
            pltpu.VMEM((S, chunk), jnp.int32),
            pltpu.VMEM((chunk, D), jnp.float32),
            pltpu.VMEM((chunk, D), jnp.float32),
            pltpu.VMEM((CB, D, 129), jnp.float32),
            pltpu.VMEM((CB, D, 129), jnp.float32),
            pltpu.SemaphoreType.DMA((2,)),
            pltpu.SemaphoreType.DMA((2,)),
        ],
        compiler_params=pltpu.CompilerParams(
            use_tc_tiling_on_sc=False, needs_layout_passes=False
        ),
    )
    def gather_kernel(table_hbm, idx_hbm, out_hbm, idx_v, r0, r1, t0, t1,
                      gsem, ssem):
        rows = [r0, r1]
        rowsT = [t0, t1]
        wid = lax.axis_index("s") * _NC + lax.axis_index("c")
        col0 = wid * chunk

        pltpu.sync_copy(idx_hbm.at[:, pl.ds(col0, chunk)], idx_v)

        lane16 = lax.iota(jnp.int32, 16)

        def gather_copy(s, b):
            return pltpu.make_async_copy(
                table_hbm.at[idx_v.at[s]], rows[b], gsem.at[b]
            )

        def store_copies(s, b):
            return [
                pltpu.make_async_copy(
                    rowsT[b].at[:, pl.ds(r * 8, 8), pl.ds(0, 128)],
                    out_hbm.at[s, r, pl.ds(wid * CB, CB)],
                    ssem.at[b],
                )
                for r in range(R)
            ]

        dlo = lane16
        dhi = lane16 + 16

        def transpose_chunk(b):
            src, dst = rows[b], rowsT[b]

            def body(g, carry):
                j0 = g * 16
                cb = g // 8
                cbv = jnp.full((16,), 0, jnp.int32) + cb
                for t in range(16):
                    j = j0 + t
                    lv = jnp.full((16,), 0, jnp.int32) + (j - cb * 128)
                    lo = src[j, pl.ds(0, 16)]
                    hi = src[j, pl.ds(16, 16)]
                    plsc.store_scatter(dst, [cbv, dlo, lv], lo)
                    plsc.store_scatter(dst, [cbv, dhi, lv], hi)
                return carry

            lax.fori_loop(0, NG, body, 0)

        gather_copy(0, 0).start()

        def outer(k, carry):
            for sub in range(2):
                s = 2 * k + sub
                b = sub
                gather_copy(s, b).wait()

                @pl.when(s + 1 < S)
                def _():
                    gather_copy(s + 1, 1 - b).start()

                @pl.when(k >= 1)
                def _():
                    for cp in store_copies(s - 2, b):
                        cp.wait()

                transpose_chunk(b)
                for cp in store_copies(s, b):
                    cp.start()
            return carry

        lax.fori_loop(0, S // 2, outer, 0)
        for sub in range(2):
            for cp in store_copies(S - 2 + sub, sub):
                cp.wait()

    return gather_kernel


def kernel(x, table):
    Bm, S = x.shape
    V, D = table.shape
    xt = jnp.swapaxes(x, 0, 1).astype(jnp.int32)
    y = _make_gather(S, Bm, V, D)(table, xt)
    return jnp.transpose(y, (2, 4, 0, 1, 3)).reshape(Bm, S, D)
